# Initial kernel scaffold; baseline (speedup 1.0000x reference)
import jax
import jax.numpy as jnp
from jax.experimental import pallas as pl


def kernel(hidden_states, attention_mask, num_tokens, pre_params, post_params, cif_w, cif_b, proj_w, proj_b, lm_w):
    raise NotImplementedError("write your pallas kernel here")


# speedup vs baseline: 11.3388x; 11.3388x over previous
import jax
import jax.numpy as jnp
from jax.experimental import pallas as pl
from jax.experimental.pallas import tpu as pltpu

D = 1280
FFN = 5120
HEADS = 8
HD = D // HEADS
T = 1500
MAX_TOKENS = 150
MT_PAD = 160
EPS = 1e-5
NEG = -1e9
VMEM_LIM = 57 * 1024 * 1024


def _params(n_par=1):
    return pltpu.CompilerParams(
        dimension_semantics=("parallel",) * n_par,
        vmem_limit_bytes=VMEM_LIM,
    )


def _fused_matmul(x, w, b, *, ln=None, gelu=False, res=None, bm, name,
                  extra_lastcol=False):
    M, K = x.shape
    N = w.shape[0]
    assert M % bm == 0, (M, bm)
    nb = M // bm
    args = [x, w, b.reshape(1, N)]
    in_specs = [
        pl.BlockSpec((bm, K), lambda i: (i, 0)),
        pl.BlockSpec((N, K), lambda i: (0, 0)),
        pl.BlockSpec((1, N), lambda i: (0, 0)),
    ]
    if ln is not None:
        s, sb = ln
        args += [s.reshape(1, K), sb.reshape(1, K)]
        in_specs += [pl.BlockSpec((1, K), lambda i: (0, 0)),
                     pl.BlockSpec((1, K), lambda i: (0, 0))]
    if res is not None:
        args.append(res)
        in_specs.append(pl.BlockSpec((bm, N), lambda i: (i, 0)))

    def body(*refs):
        x_ref, w_ref, b_ref = refs[0], refs[1], refs[2]
        i = 3
        xb = x_ref[...]
        if ln is not None:
            mu = jnp.mean(xb, axis=-1, keepdims=True)
            xc = xb - mu
            var = jnp.mean(xc * xc, axis=-1, keepdims=True)
            xb = xc * jax.lax.rsqrt(var + EPS) * refs[i][...] + refs[i + 1][...]
            i += 2
        acc = jax.lax.dot_general(
            xb, w_ref[...], (((1,), (1,)), ((), ())),
            preferred_element_type=jnp.float32)
        acc = acc + b_ref[...]
        if gelu:
            acc = 0.5 * acc * (1.0 + jax.lax.erf(acc * (2.0 ** -0.5)))
        if res is not None:
            acc = acc + refs[i][...]
            i += 1
        if extra_lastcol:
            refs[-2][...] = acc
            refs[-1][...] = acc[:, N - 1:N]
        else:
            refs[-1][...] = acc

    if extra_lastcol:
        out_shape = [jax.ShapeDtypeStruct((M, N), jnp.float32),
                     jax.ShapeDtypeStruct((M, 1), jnp.float32)]
        out_specs = [pl.BlockSpec((bm, N), lambda i: (i, 0)),
                     pl.BlockSpec((bm, 1), lambda i: (i, 0))]
    else:
        out_shape = jax.ShapeDtypeStruct((M, N), jnp.float32)
        out_specs = pl.BlockSpec((bm, N), lambda i: (i, 0))
    return pl.pallas_call(
        body, grid=(nb,), in_specs=in_specs, out_specs=out_specs,
        out_shape=out_shape, compiler_params=_params(), name=name,
    )(*args)


def _matmul_ntile(x, w, b, *, bn, name):
    M, K = x.shape
    N = w.shape[0]
    args = [x, w] + ([] if b is None else [b.reshape(1, N)])

    def body(*refs):
        acc = jax.lax.dot_general(
            refs[0][...], refs[1][...], (((1,), (1,)), ((), ())),
            preferred_element_type=jnp.float32)
        if b is not None:
            acc = acc + refs[2][...]
        refs[-1][...] = acc

    in_specs = [pl.BlockSpec((M, K), lambda j: (0, 0)),
                pl.BlockSpec((bn, K), lambda j: (j, 0))]
    if b is not None:
        in_specs.append(pl.BlockSpec((1, bn), lambda j: (0, j)))
    return pl.pallas_call(
        body, grid=(N // bn,), in_specs=in_specs,
        out_specs=pl.BlockSpec((M, bn), lambda j: (0, j)),
        out_shape=jax.ShapeDtypeStruct((M, N), jnp.float32),
        compiler_params=_params(), name=name,
    )(*args)


def _attention(q, k, v, nt=None, name="attn"):
    G, L, Hd = q.shape
    scale = HD ** -0.5

    def body(*refs):
        if nt is None:
            q_ref, k_ref, v_ref, o_ref = refs
        else:
            q_ref, k_ref, v_ref, nt_ref, o_ref = refs
        qb = q_ref[0] * scale
        s = jax.lax.dot_general(
            qb, k_ref[0], (((1,), (1,)), ((), ())),
            preferred_element_type=jnp.float32)
        if nt is not None:
            bidx = pl.program_id(0) // HEADS
            n = jnp.maximum(nt_ref[bidx], 1)
            col = jax.lax.broadcasted_iota(jnp.int32, (L, L), 1)
            s = s + jnp.where(col < n, 0.0, NEG)
        m = jnp.max(s, axis=-1, keepdims=True)
        e = jnp.exp(s - m)
        p = e / jnp.sum(e, axis=-1, keepdims=True)
        o_ref[0] = jax.lax.dot_general(
            p, v_ref[0], (((1,), (0,)), ((), ())),
            preferred_element_type=jnp.float32)

    args = [q, k, v]
    in_specs = [pl.BlockSpec((1, L, Hd), lambda i: (i, 0, 0))] * 3
    if nt is not None:
        args.append(nt)
        in_specs.append(pl.BlockSpec(memory_space=pltpu.SMEM))
    return pl.pallas_call(
        body, grid=(G,), in_specs=in_specs,
        out_specs=pl.BlockSpec((1, L, Hd), lambda i: (i, 0, 0)),
        out_shape=jax.ShapeDtypeStruct((G, L, Hd), jnp.float32),
        compiler_params=_params(), name=name,
    )(*args)


def _alphas_kernel(hl, mask, nt):
    B = hl.shape[0]

    def body(hl_ref, m_ref, nt_ref, a_ref, s_ref):
        araw = jax.nn.sigmoid(hl_ref[...]) * m_ref[...]
        asum = jnp.sum(araw, axis=-1, keepdims=True)
        rowi = jax.lax.broadcasted_iota(jnp.int32, (B, 1), 0)
        ntf = jnp.zeros((B, 1), jnp.float32)
        for i in range(B):
            ntf = jnp.where(rowi == i,
                            jnp.maximum(nt_ref[i], 1).astype(jnp.float32), ntf)
        a_ref[...] = araw * (ntf / asum)
        s_ref[...] = asum

    return pl.pallas_call(
        body,
        in_specs=[pl.BlockSpec((B, T), lambda: (0, 0)),
                  pl.BlockSpec((B, T), lambda: (0, 0)),
                  pl.BlockSpec(memory_space=pltpu.SMEM)],
        out_specs=[pl.BlockSpec((B, T), lambda: (0, 0)),
                   pl.BlockSpec((B, 1), lambda: (0, 0))],
        out_shape=[jax.ShapeDtypeStruct((B, T), jnp.float32),
                   jax.ShapeDtypeStruct((B, 1), jnp.float32)],
        compiler_params=pltpu.CompilerParams(vmem_limit_bytes=VMEM_LIM),
        name="alphas",
    )(hl, mask, nt)


def _cif_scan(aT):
    Tt, B = aT.shape

    def body(a_ref, o_ref):
        def step(t, carry):
            integ, fcnt = carry
            a_t = a_ref[pl.ds(t, 1), :]
            integ2 = integ + a_t
            fire = integ2 >= 1.0
            integ3 = jnp.where(fire, integ2 - 1.0, integ2)
            f2 = fcnt + jnp.where(fire, 1.0, 0.0)
            o_ref[t] = jnp.concatenate([integ3, f2], axis=1)
            return (integ3, f2)
        jax.lax.fori_loop(
            0, Tt, step,
            (jnp.zeros((1, B), jnp.float32), jnp.zeros((1, B), jnp.float32)))

    return pl.pallas_call(
        body,
        in_specs=[pl.BlockSpec((Tt, B), lambda: (0, 0))],
        out_specs=pl.BlockSpec((Tt, 1, 2 * B), lambda: (0, 0, 0)),
        out_shape=jax.ShapeDtypeStruct((Tt, 1, 2 * B), jnp.float32),
        compiler_params=pltpu.CompilerParams(vmem_limit_bytes=VMEM_LIM),
        name="cif_scan",
    )(aT)


def _cif_weights_matmul(scan2d, aT, nt, h):
    B = h.shape[0]

    def body2(sc_ref, a_ref, nt_ref, h_ref, o_ref):
        b = pl.program_id(0)
        sc = sc_ref[...]
        lane8 = jax.lax.broadcasted_iota(jnp.int32, (2 * B, MT_PAD), 0)
        e_int = jnp.where(lane8 == b, 1.0, 0.0)
        e_f = jnp.where(lane8 == B + b, 1.0, 0.0)
        laneB = jax.lax.broadcasted_iota(jnp.int32, (B, MT_PAD), 0)
        e_a = jnp.where(laneB == b, 1.0, 0.0)
        integ = jax.lax.dot_general(sc, e_int, (((1,), (0,)), ((), ())),
                                    preferred_element_type=jnp.float32)
        fcnt = jax.lax.dot_general(sc, e_f, (((1,), (0,)), ((), ())),
                                   preferred_element_type=jnp.float32)
        a = jax.lax.dot_general(a_ref[...], e_a, (((1,), (0,)), ((), ())),
                                preferred_element_type=jnp.float32)
        zrow = jnp.zeros((1, MT_PAD), jnp.float32)
        ipre = jnp.concatenate([zrow, integ[:-1, :]], axis=0)
        fpre = jnp.concatenate([zrow, fcnt[:-1, :]], axis=0)
        fire = fcnt > fpre
        an = 1.0 - ipre
        val1 = jnp.where(fire, an, a)
        rowi = jax.lax.broadcasted_iota(jnp.int32, (T, MT_PAD), 0)
        val2 = jnp.where(fire & (rowi < T - 1), a - an, 0.0)
        ntf1 = jnp.maximum(nt_ref[b], 1).astype(jnp.float32) - 1.0
        m1 = jnp.minimum(fpre, ntf1)
        m2 = jnp.minimum(fcnt, ntf1)
        tok = jax.lax.broadcasted_iota(
            jnp.int32, (T, MT_PAD), 1).astype(jnp.float32)
        w = jnp.where(tok == m1, val1, 0.0) + jnp.where(tok == m2, val2, 0.0)
        o_ref[0] = jax.lax.dot_general(
            w, h_ref[0], (((0,), (0,)), ((), ())),
            preferred_element_type=jnp.float32)

    return pl.pallas_call(
        body2, grid=(B,),
        in_specs=[pl.BlockSpec((T, 2 * B), lambda i: (0, 0)),
                  pl.BlockSpec((T, B), lambda i: (0, 0)),
                  pl.BlockSpec(memory_space=pltpu.SMEM),
                  pl.BlockSpec((1, T, D), lambda i: (i, 0, 0))],
        out_specs=pl.BlockSpec((1, MT_PAD, D), lambda i: (i, 0, 0)),
        out_shape=jax.ShapeDtypeStruct((B, MT_PAD, D), jnp.float32),
        compiler_params=_params(), name="cif_weights",
    )(scan2d, aT, nt, h)


def _split_heads(x, B, L):
    x = x.reshape(B, L, 3, HEADS, HD).transpose(2, 0, 3, 1, 4)
    return (x[0].reshape(B * HEADS, L, HD),
            x[1].reshape(B * HEADS, L, HD),
            x[2].reshape(B * HEADS, L, HD))


def _merge_heads(o, B, L):
    return o.reshape(B, HEADS, L, HD).transpose(0, 2, 1, 3).reshape(B * L, D)


def _encoder_layer(x2d, p, B, L, bm, bm_mlp, nt=None, want_lastcol=False,
                   tag=""):
    M = B * L
    wqkv = jnp.concatenate([p['q_w'], p['k_w'], p['v_w']], axis=0)
    bqkv = jnp.concatenate([p['q_b'], jnp.zeros((D,), jnp.float32), p['v_b']])
    qkv = _fused_matmul(x2d, wqkv, bqkv, ln=(p['ln1_s'], p['ln1_b']),
                        bm=bm, name="qkv" + tag)
    q3, k3, v3 = _split_heads(qkv, B, L)
    o3 = _attention(q3, k3, v3, nt=nt, name="attn" + tag)
    attn2d = _merge_heads(o3, B, L)
    x1 = _fused_matmul(attn2d, p['o_w'], p['o_b'], res=x2d, bm=bm,
                       name="oproj" + tag)
    g = _fused_matmul(x1, p['fc1_w'], p['fc1_b'], ln=(p['ln2_s'], p['ln2_b']),
                      gelu=True, bm=bm_mlp, name="fc1" + tag)
    return _fused_matmul(g, p['fc2_w'], p['fc2_b'], res=x1, bm=bm_mlp,
                         name="fc2" + tag, extra_lastcol=want_lastcol)


def kernel(hidden_states, attention_mask, num_tokens, pre_params, post_params,
           cif_w, cif_b, proj_w, proj_b, lm_w):
    B, L, _ = hidden_states.shape
    x2d = hidden_states.reshape(B * L, D)

    h, hl = _encoder_layer(x2d, pre_params, B, L, bm=600, bm_mlp=400,
                           want_lastcol=True, tag="_pre")

    alphas, asum = _alphas_kernel(hl.reshape(B, L), attention_mask, num_tokens)
    aT = alphas.T
    scan2d = _cif_scan(aT).reshape(T, 2 * B)
    cif = _cif_weights_matmul(scan2d, aT, num_tokens, h.reshape(B, L, D))

    cif_w_pad = jnp.concatenate(
        [cif_w, jnp.zeros((D, 1), jnp.float32)], axis=1)
    h2full = _fused_matmul(cif.reshape(B * MT_PAD, D), cif_w_pad, cif_b,
                           bm=320, name="cifproj")
    h2 = h2full.reshape(B, MT_PAD, D)[:, :MAX_TOKENS, :].reshape(
        B * MAX_TOKENS, D)

    h3 = _encoder_layer(h2, post_params, B, MAX_TOKENS, bm=600, bm_mlp=200,
                        nt=num_tokens, tag="_post")

    logits = _matmul_ntile(h3, lm_w, None, bn=3200, name="lmhead")
    hid = _fused_matmul(h3, proj_w, proj_b, bm=200, name="proj")

    nt = jnp.maximum(num_tokens, 1)
    out_mask = (jnp.arange(MAX_TOKENS)[None, :] < nt[:, None]).astype(jnp.int32)
    return (hid.reshape(B, MAX_TOKENS, -1), out_mask,
            logits.reshape(B, MAX_TOKENS, -1), alphas, asum.reshape(B))
